# baseline (speedup 1.0000x reference)
import functools

import jax
import jax.numpy as jnp
from jax import lax
from jax.experimental import pallas as pl
from jax.experimental.pallas import tpu as pltpu
from jax.experimental.pallas import tpu_sc as plsc

H = 64
N_NODES = 10000
NPAD = 10112
E_BIG = 160000
EPAD = 163840
E_SMALL = 10000
ECPAD = 10240
NC, NS, LANES = 2, 16, 16
NW = NC * NS
DUMMY_DST = N_NODES
ROWS_PER_SUB = NPAD // NS

_GELU_C = 1.5957691216057308
_GELU_A = 0.044715


def _mm_kern(x_ref, w_ref, o_ref):
    o_ref[...] = jnp.dot(x_ref[...], w_ref[...],
                         preferred_element_type=jnp.float32)


def _mm_bias_kern(x_ref, w_ref, b_ref, o_ref):
    o_ref[...] = jnp.dot(x_ref[...], w_ref[...],
                         preferred_element_type=jnp.float32) + b_ref[...]


def _tc_matmul(x, w, b=None, tm=512):
    m, k = x.shape
    n = w.shape[1]
    tm = min(tm, m)
    grid = (m // tm,)
    in_specs = [pl.BlockSpec((tm, k), lambda i: (i, 0)),
                pl.BlockSpec((k, n), lambda i: (0, 0))]
    args = [x, w]
    kern = _mm_kern
    if b is not None:
        in_specs.append(pl.BlockSpec((1, n), lambda i: (0, 0)))
        args.append(b.reshape(1, n))
        kern = _mm_bias_kern
    return pl.pallas_call(
        kern, grid=grid, in_specs=in_specs,
        out_specs=pl.BlockSpec((tm, n), lambda i: (i, 0)),
        out_shape=jax.ShapeDtypeStruct((m, n), jnp.float32),
    )(*args)


def _enc_kern(x_ref, w1_ref, b1_ref, w2_ref, b2_ref, o_ref):
    h = jnp.dot(x_ref[...], w1_ref[...],
                preferred_element_type=jnp.float32) + b1_ref[...]
    h = jax.nn.gelu(h)
    o_ref[...] = jnp.dot(h, w2_ref[...],
                         preferred_element_type=jnp.float32) + b2_ref[...]


def _tc_encode(x, mlp, tm=512):
    m, k = x.shape
    h1 = mlp[0]['W'].shape[1]
    h2 = mlp[1]['W'].shape[1]
    tm = min(tm, m)
    return pl.pallas_call(
        _enc_kern, grid=(m // tm,),
        in_specs=[pl.BlockSpec((tm, k), lambda i: (i, 0)),
                  pl.BlockSpec((k, h1), lambda i: (0, 0)),
                  pl.BlockSpec((1, h1), lambda i: (0, 0)),
                  pl.BlockSpec((h1, h2), lambda i: (0, 0)),
                  pl.BlockSpec((1, h2), lambda i: (0, 0))],
        out_specs=pl.BlockSpec((tm, h2), lambda i: (i, 0)),
        out_shape=jax.ShapeDtypeStruct((m, h2), jnp.float32),
    )(x, mlp[0]['W'], mlp[0]['b'].reshape(1, h1),
      mlp[1]['W'], mlp[1]['b'].reshape(1, h2))


def _conv_tail_kern(x_ref, pa_ref, pb_ref, w2a_ref, w2b_ref, g_ref, b_ref,
                    o_ref):
    agg_a = pa_ref[0] + pa_ref[1]
    agg_b = pb_ref[0] + pb_ref[1]
    a = (jnp.dot(agg_a, w2a_ref[...], preferred_element_type=jnp.float32)
         + jnp.dot(agg_b, w2b_ref[...], preferred_element_type=jnp.float32))
    t = x_ref[...] + a
    mu = jnp.mean(t, axis=-1, keepdims=True)
    v = jnp.mean((t - mu) ** 2, axis=-1, keepdims=True)
    o_ref[...] = (t - mu) * lax.rsqrt(v + 1e-5) * g_ref[...] + b_ref[...]


def _tc_conv_tail(x, part_a, part_b, w2a, w2b, ln, tm=128):
    m = x.shape[0]
    return pl.pallas_call(
        _conv_tail_kern, grid=(m // tm,),
        in_specs=[pl.BlockSpec((tm, H), lambda i: (i, 0)),
                  pl.BlockSpec((2, tm, H), lambda i: (0, i, 0)),
                  pl.BlockSpec((2, tm, H), lambda i: (0, i, 0)),
                  pl.BlockSpec((H, H), lambda i: (0, 0)),
                  pl.BlockSpec((H, H), lambda i: (0, 0)),
                  pl.BlockSpec((1, H), lambda i: (0, 0)),
                  pl.BlockSpec((1, H), lambda i: (0, 0))],
        out_specs=pl.BlockSpec((tm, H), lambda i: (i, 0)),
        out_shape=jax.ShapeDtypeStruct((m, H), jnp.float32),
    )(x, part_a, part_b, w2a, w2b,
      ln['g'].reshape(1, H), ln['b'].reshape(1, H))


def _eu_tail_kern(pre_ref, ep_ref, w2_ref, b2_ref, o_ref):
    g = jax.nn.gelu(pre_ref[...])
    o_ref[...] = ep_ref[...] + jnp.dot(
        g, w2_ref[...], preferred_element_type=jnp.float32) + b2_ref[...]


def _tc_eu_tail(pre, ep, w2, b2, tm=512):
    m = pre.shape[0]
    return pl.pallas_call(
        _eu_tail_kern, grid=(m // tm,),
        in_specs=[pl.BlockSpec((tm, H), lambda i: (i, 0)),
                  pl.BlockSpec((tm, H), lambda i: (i, 0)),
                  pl.BlockSpec((H, H), lambda i: (0, 0)),
                  pl.BlockSpec((1, H), lambda i: (0, 0))],
        out_specs=pl.BlockSpec((tm, H), lambda i: (i, 0)),
        out_shape=jax.ShapeDtypeStruct((m, H), jnp.float32),
    )(pre, ep, w2, b2.reshape(1, H))


def _head_kern(x_ref, h_ref, wi_ref, bi_ref, wh_ref, bh_ref,
               wgx_ref, wgh_ref, bg_ref, v1_ref, c1_ref, v2_ref, c2_ref,
               hn_ref, o_ref):
    x = x_ref[...]
    h = h_ref[...]
    gx = jnp.dot(x, wi_ref[...], preferred_element_type=jnp.float32) \
        + bi_ref[...]
    gh = jnp.dot(h, wh_ref[...], preferred_element_type=jnp.float32) \
        + bh_ref[...]
    r = jax.nn.sigmoid(gx[:, 0:H] + gh[:, 0:H])
    z = jax.nn.sigmoid(gx[:, H:2 * H] + gh[:, H:2 * H])
    n = jnp.tanh(gx[:, 2 * H:3 * H] + r * gh[:, 2 * H:3 * H])
    hn = (1.0 - z) * n + z * h
    g = jax.nn.sigmoid(
        jnp.dot(x, wgx_ref[...], preferred_element_type=jnp.float32)
        + jnp.dot(hn, wgh_ref[...], preferred_element_type=jnp.float32)
        + bg_ref[...])
    xo = (1.0 - g) * x + g * hn
    d = jax.nn.gelu(
        jnp.dot(xo, v1_ref[...], preferred_element_type=jnp.float32)
        + c1_ref[...])
    o_ref[...] = jnp.dot(d, v2_ref[...],
                         preferred_element_type=jnp.float32) + c2_ref[...]
    hn_ref[...] = hn


def _tc_head(x, h, gru, gate, dec, tm=128):
    m = x.shape[0]
    full = lambda shape: pl.BlockSpec(shape, lambda i: tuple(0 for _ in shape))
    return pl.pallas_call(
        _head_kern, grid=(m // tm,),
        in_specs=[pl.BlockSpec((tm, H), lambda i: (i, 0)),
                  pl.BlockSpec((tm, H), lambda i: (i, 0)),
                  full((H, 3 * H)), full((1, 3 * H)),
                  full((H, 3 * H)), full((1, 3 * H)),
                  full((H, H)), full((H, H)), full((1, H)),
                  full((H, H)), full((1, H)), full((H, 1)), full((1, 1))],
        out_specs=[pl.BlockSpec((tm, H), lambda i: (i, 0)),
                   pl.BlockSpec((tm, 1), lambda i: (i, 0))],
        out_shape=[jax.ShapeDtypeStruct((m, H), jnp.float32),
                   jax.ShapeDtypeStruct((m, 1), jnp.float32)],
    )(x, h, gru['Wi'], gru['bi'].reshape(1, 3 * H),
      gru['Wh'], gru['bh'].reshape(1, 3 * H),
      gate['W'][:H], gate['W'][H:], gate['b'].reshape(1, H),
      dec[0]['W'], dec[0]['b'].reshape(1, H),
      dec[1]['W'], dec[1]['b'].reshape(1, 1))


def _sc_gelu16(v):
    t2 = _GELU_C * (v + _GELU_A * v * v * v)
    t2 = jnp.minimum(t2, 60.0)
    e = jnp.exp(t2)
    d = 1.0 + e
    q = 1.0 / d
    q = q * (2.0 - d * q)
    return v * (1.0 - q)


def _make_sc_conv(nchunks, chunk, with_gelu, out_is_acc, epad):
    per_tile = epad // NW
    mesh = plsc.VectorSubcoreMesh(core_axis_name="c", subcore_axis_name="s",
                                  num_cores=NC, num_subcores=NS)
    if out_is_acc:
        out_type = jax.ShapeDtypeStruct((NC, NPAD, H), jnp.float32)
    else:
        out_type = jax.ShapeDtypeStruct((epad, H), jnp.float32)
    scratch = [
        pltpu.VMEM((nchunks, chunk), jnp.int32),
        pltpu.VMEM((nchunks, chunk), jnp.int32),
        pltpu.VMEM((chunk, H), jnp.float32),
        pltpu.VMEM((chunk, H), jnp.float32),
        pltpu.VMEM((chunk, H), jnp.float32),
        pltpu.VMEM((chunk, H), jnp.float32),
    ]
    if out_is_acc:
        scratch.append(pltpu.VMEM((ROWS_PER_SUB, H), jnp.float32))
        scratch.append(pltpu.VMEM_SHARED((NPAD, H), jnp.float32))

    def body(s_hbm, d_hbm, em_hbm, src_hbm, dst_hbm, out_hbm, *scr):
        if out_is_acc:
            srcv, dstv, srows, drows, emrows, zrows, zbuf, acc = scr
        else:
            srcv, dstv, srows, drows, emrows, zrows = scr
        c = lax.axis_index("c")
        s = lax.axis_index("s")
        wid = c * NS + s
        row0 = wid * nchunks
        base = wid * per_tile
        pltpu.sync_copy(src_hbm.at[pl.ds(row0, nchunks)], srcv)
        pltpu.sync_copy(dst_hbm.at[pl.ds(row0, nchunks)], dstv)

        if out_is_acc:
            def zero_row(r, _):
                for g in range(H // LANES):
                    zbuf[r, pl.ds(g * LANES, LANES)] = jnp.zeros(
                        (LANES,), jnp.float32)
                return 0
            lax.fori_loop(0, ROWS_PER_SUB, zero_row, 0)
            pltpu.sync_copy(zbuf,
                            acc.at[pl.ds(s * ROWS_PER_SUB, ROWS_PER_SUB)])
            plsc.subcore_barrier()

        def chunk_step(j, _):
            pltpu.sync_copy(s_hbm.at[srcv.at[j]], srows)
            pltpu.sync_copy(d_hbm.at[dstv.at[j]], drows)
            pltpu.sync_copy(em_hbm.at[pl.ds(base + j * chunk, chunk)], emrows)

            def row_step(r, _):
                for g in range(H // LANES):
                    sl = pl.ds(g * LANES, LANES)
                    v = srows[r, sl] + drows[r, sl] + emrows[r, sl]
                    if with_gelu:
                        v = _sc_gelu16(v)
                    zrows[r, sl] = v
                return 0
            lax.fori_loop(0, chunk, row_step, 0)

            if out_is_acc:
                pltpu.sync_copy(zrows, acc.at[dstv.at[j]], add=True)
            else:
                pltpu.sync_copy(zrows,
                                out_hbm.at[pl.ds(base + j * chunk, chunk)])
            return 0
        lax.fori_loop(0, nchunks, chunk_step, 0)

        if out_is_acc:
            plsc.subcore_barrier()
            sl = pl.ds(s * ROWS_PER_SUB, ROWS_PER_SUB)
            pltpu.sync_copy(acc.at[sl], out_hbm.at[c].at[sl])

    return pl.kernel(body, out_type=out_type, mesh=mesh,
                     scratch_types=scratch,
                     compiler_params=pltpu.CompilerParams(
                         use_tc_tiling_on_sc=False))


@functools.cache
def _sc_kernels():
    return {
        'conv_big': _make_sc_conv(40, 128, True, True, EPAD),
        'conv_small': _make_sc_conv(5, 64, True, True, ECPAD),
        'eu_big': _make_sc_conv(40, 128, False, False, EPAD),
    }


def _pad_rows(x, rows):
    return jnp.pad(x, ((0, rows - x.shape[0]), (0, 0)))


def _pad_idx(ei, epad):
    src = jnp.pad(ei[0], (0, epad - ei.shape[1]))
    dst = jnp.pad(ei[1], (0, epad - ei.shape[1]),
                  constant_values=DUMMY_DST)
    chunk = 128 if epad == EPAD else 64
    return src.reshape(-1, chunk), dst.reshape(-1, chunk)


def kernel(params, x_1d, x_2d, edge_index_pipe, edge_attr_pipe,
           edge_index_surface, edge_attr_surface, edge_index_c12,
           edge_attr_c12, edge_index_c21, edge_attr_c21, h_1d, h_2d):
    sck = _sc_kernels()

    src_p, dst_p = _pad_idx(edge_index_pipe, EPAD)
    src_s, dst_s = _pad_idx(edge_index_surface, EPAD)
    src_12, dst_12 = _pad_idx(edge_index_c12, ECPAD)
    src_21, dst_21 = _pad_idx(edge_index_c21, ECPAD)

    x1 = _tc_encode(_pad_rows(x_1d, NPAD), params['enc_1d'], tm=128)
    x2 = _tc_encode(_pad_rows(x_2d, NPAD), params['enc_2d'], tm=128)
    ep = _tc_encode(_pad_rows(edge_attr_pipe, EPAD), params['enc_pipe'])
    es = _tc_encode(_pad_rows(edge_attr_surface, EPAD), params['enc_surf'])
    e12 = _tc_encode(_pad_rows(edge_attr_c12, ECPAD), params['enc_c12'],
                     tm=128)
    e21 = _tc_encode(_pad_rows(edge_attr_c21, ECPAD), params['enc_c21'],
                     tm=128)

    for lp in params['layers']:
        wp1, wp2 = lp['conv_pipe'][0], lp['conv_pipe'][1]
        ws1, ws2 = lp['conv_surf'][0], lp['conv_surf'][1]
        w12_1, w12_2 = lp['conv_c12'][0], lp['conv_c12'][1]
        w21_1, w21_2 = lp['conv_c21'][0], lp['conv_c21'][1]

        p1 = _tc_matmul(x1, jnp.concatenate(
            [wp1['W'][:H], wp1['W'][2 * H:],
             w12_1['W'][:H], w21_1['W'][2 * H:]], axis=1), tm=128)
        p2 = _tc_matmul(x2, jnp.concatenate(
            [ws1['W'][:H], ws1['W'][2 * H:],
             w21_1['W'][:H], w12_1['W'][2 * H:]], axis=1), tm=128)

        em_p = _tc_matmul(ep, wp1['W'][H:2 * H], b=wp1['b'])
        em_s = _tc_matmul(es, ws1['W'][H:2 * H], b=ws1['b'])
        em_12 = _tc_matmul(e12, w12_1['W'][H:2 * H], b=w12_1['b'], tm=128)
        em_21 = _tc_matmul(e21, w21_1['W'][H:2 * H], b=w21_1['b'], tm=128)

        part_p = sck['conv_big'](p1[:, 0:H], p1[:, H:2 * H], em_p,
                                 src_p, dst_p)
        part_s = sck['conv_big'](p2[:, 0:H], p2[:, H:2 * H], em_s,
                                 src_s, dst_s)
        part_12 = sck['conv_small'](p1[:, 2 * H:3 * H], p2[:, 3 * H:4 * H],
                                    em_12, src_12, dst_12)
        part_21 = sck['conv_small'](p2[:, 2 * H:3 * H], p1[:, 3 * H:4 * H],
                                    em_21, src_21, dst_21)

        x1 = _tc_conv_tail(x1, part_p, part_21, wp2['W'], w21_2['W'],
                           lp['ln_1d'])
        x2 = _tc_conv_tail(x2, part_s, part_12, ws2['W'], w12_2['W'],
                           lp['ln_2d'])

        eup1, eup2 = lp['eu_pipe'][0], lp['eu_pipe'][1]
        eus1, eus2 = lp['eu_surf'][0], lp['eu_surf'][1]
        pu1 = _tc_matmul(x1, jnp.concatenate(
            [eup1['W'][:H], eup1['W'][2 * H:]], axis=1), tm=128)
        pu2 = _tc_matmul(x2, jnp.concatenate(
            [eus1['W'][:H], eus1['W'][2 * H:]], axis=1), tm=128)
        emu_p = _tc_matmul(ep, eup1['W'][H:2 * H], b=eup1['b'])
        emu_s = _tc_matmul(es, eus1['W'][H:2 * H], b=eus1['b'])
        pre_p = sck['eu_big'](pu1[:, 0:H], pu1[:, H:2 * H], emu_p,
                              src_p, dst_p)
        pre_s = sck['eu_big'](pu2[:, 0:H], pu2[:, H:2 * H], emu_s,
                              src_s, dst_s)
        ep = _tc_eu_tail(pre_p, ep, eup2['W'], eup2['b'])
        es = _tc_eu_tail(pre_s, es, eus2['W'], eus2['b'])

    h1n, out1 = _tc_head(x1, _pad_rows(h_1d, NPAD), params['gru_1d'],
                         params['gate_1d'], params['dec_1d'])
    h2n, out2 = _tc_head(x2, _pad_rows(h_2d, NPAD), params['gru_2d'],
                         params['gate_2d'], params['dec_2d'])

    return (out1[:N_NODES], out2[:N_NODES],
            h1n[:N_NODES], h2n[:N_NODES])


# speedup vs baseline: 1.1491x; 1.1491x over previous
import functools

import jax
import jax.numpy as jnp
from jax import lax
from jax.experimental import pallas as pl
from jax.experimental.pallas import tpu as pltpu
from jax.experimental.pallas import tpu_sc as plsc

H = 64
N_NODES = 10000
NPAD = 10112
EPAD = 163840
ECPAD = 10240
NC, NS, LANES = 2, 16, 16
NW = NC * NS
DUMMY_DST = N_NODES
ROWS_PER_SUB = NPAD // NS


def _enc_kern(x_ref, w1_ref, b1_ref, w2_ref, b2_ref, o_ref):
    h = jnp.dot(x_ref[...], w1_ref[...],
                preferred_element_type=jnp.float32) + b1_ref[...]
    h = jax.nn.gelu(h)
    o_ref[...] = jnp.dot(h, w2_ref[...],
                         preferred_element_type=jnp.float32) + b2_ref[...]


def _tc_encode(x, mlp, tm=512):
    m, k = x.shape
    h1 = mlp[0]['W'].shape[1]
    h2 = mlp[1]['W'].shape[1]
    tm = min(tm, m)
    return pl.pallas_call(
        _enc_kern, grid=(m // tm,),
        in_specs=[pl.BlockSpec((tm, k), lambda i: (i, 0)),
                  pl.BlockSpec((k, h1), lambda i: (0, 0)),
                  pl.BlockSpec((1, h1), lambda i: (0, 0)),
                  pl.BlockSpec((h1, h2), lambda i: (0, 0)),
                  pl.BlockSpec((1, h2), lambda i: (0, 0))],
        out_specs=pl.BlockSpec((tm, h2), lambda i: (i, 0)),
        out_shape=jax.ShapeDtypeStruct((m, h2), jnp.float32),
    )(x, mlp[0]['W'], mlp[0]['b'].reshape(1, h1),
      mlp[1]['W'], mlp[1]['b'].reshape(1, h2))


def _edge_mlp_kern(residual, xs_ref, ee_ref, xd_ref, w1_ref, b1_ref,
                   w2_ref, b2_ref, o_ref):
    cat = jnp.concatenate([xs_ref[...], ee_ref[...], xd_ref[...]], axis=-1)
    h = jnp.dot(cat, w1_ref[...],
                preferred_element_type=jnp.float32) + b1_ref[...]
    h = jax.nn.gelu(h)
    m = jnp.dot(h, w2_ref[...],
                preferred_element_type=jnp.float32) + b2_ref[...]
    if residual:
        m = ee_ref[...] + m
    o_ref[...] = m


def _tc_edge_mlp(xs, ee, xd, mlp, residual, tm=512):
    m = xs.shape[0]
    return pl.pallas_call(
        functools.partial(_edge_mlp_kern, residual), grid=(m // tm,),
        in_specs=[pl.BlockSpec((tm, H), lambda i: (i, 0)),
                  pl.BlockSpec((tm, H), lambda i: (i, 0)),
                  pl.BlockSpec((tm, H), lambda i: (i, 0)),
                  pl.BlockSpec((3 * H, H), lambda i: (0, 0)),
                  pl.BlockSpec((1, H), lambda i: (0, 0)),
                  pl.BlockSpec((H, H), lambda i: (0, 0)),
                  pl.BlockSpec((1, H), lambda i: (0, 0))],
        out_specs=pl.BlockSpec((tm, H), lambda i: (i, 0)),
        out_shape=jax.ShapeDtypeStruct((m, H), jnp.float32),
    )(xs, ee, xd, mlp[0]['W'], mlp[0]['b'].reshape(1, H),
      mlp[1]['W'], mlp[1]['b'].reshape(1, H))


def _tail_kern(x_ref, pa_ref, pb_ref, g_ref, b_ref, o_ref):
    a = (pa_ref[0] + pa_ref[1]) + (pb_ref[0] + pb_ref[1])
    t = x_ref[...] + a
    mu = jnp.mean(t, axis=-1, keepdims=True)
    v = jnp.mean((t - mu) ** 2, axis=-1, keepdims=True)
    o_ref[...] = (t - mu) / jnp.sqrt(v + 1e-5) * g_ref[...] + b_ref[...]


def _tc_tail(x, part_a, part_b, ln, tm=128):
    m = x.shape[0]
    return pl.pallas_call(
        _tail_kern, grid=(m // tm,),
        in_specs=[pl.BlockSpec((tm, H), lambda i: (i, 0)),
                  pl.BlockSpec((2, tm, H), lambda i: (0, i, 0)),
                  pl.BlockSpec((2, tm, H), lambda i: (0, i, 0)),
                  pl.BlockSpec((1, H), lambda i: (0, 0)),
                  pl.BlockSpec((1, H), lambda i: (0, 0))],
        out_specs=pl.BlockSpec((tm, H), lambda i: (i, 0)),
        out_shape=jax.ShapeDtypeStruct((m, H), jnp.float32),
    )(x, part_a, part_b, ln['g'].reshape(1, H), ln['b'].reshape(1, H))


def _head_kern(x_ref, h_ref, wi_ref, bi_ref, wh_ref, bh_ref,
               wg_ref, bg_ref, v1_ref, c1_ref, v2_ref, c2_ref,
               hn_ref, o_ref):
    x = x_ref[...]
    h = h_ref[...]
    gx = jnp.dot(x, wi_ref[...], preferred_element_type=jnp.float32) \
        + bi_ref[...]
    gh = jnp.dot(h, wh_ref[...], preferred_element_type=jnp.float32) \
        + bh_ref[...]
    r = jax.nn.sigmoid(gx[:, 0:H] + gh[:, 0:H])
    z = jax.nn.sigmoid(gx[:, H:2 * H] + gh[:, H:2 * H])
    n = jnp.tanh(gx[:, 2 * H:3 * H] + r * gh[:, 2 * H:3 * H])
    hn = (1.0 - z) * n + z * h
    g = jax.nn.sigmoid(
        jnp.dot(jnp.concatenate([x, hn], axis=-1), wg_ref[...],
                preferred_element_type=jnp.float32) + bg_ref[...])
    xo = (1.0 - g) * x + g * hn
    d = jax.nn.gelu(
        jnp.dot(xo, v1_ref[...], preferred_element_type=jnp.float32)
        + c1_ref[...])
    o_ref[...] = jnp.dot(d, v2_ref[...],
                         preferred_element_type=jnp.float32) + c2_ref[...]
    hn_ref[...] = hn


def _tc_head(x, h, gru, gate, dec, tm=128):
    m = x.shape[0]
    full = lambda shape: pl.BlockSpec(shape, lambda i: tuple(0 for _ in shape))
    return pl.pallas_call(
        _head_kern, grid=(m // tm,),
        in_specs=[pl.BlockSpec((tm, H), lambda i: (i, 0)),
                  pl.BlockSpec((tm, H), lambda i: (i, 0)),
                  full((H, 3 * H)), full((1, 3 * H)),
                  full((H, 3 * H)), full((1, 3 * H)),
                  full((2 * H, H)), full((1, H)),
                  full((H, H)), full((1, H)), full((H, 1)), full((1, 1))],
        out_specs=[pl.BlockSpec((tm, H), lambda i: (i, 0)),
                   pl.BlockSpec((tm, 1), lambda i: (i, 0))],
        out_shape=[jax.ShapeDtypeStruct((m, H), jnp.float32),
                   jax.ShapeDtypeStruct((m, 1), jnp.float32)],
    )(x, h, gru['Wi'], gru['bi'].reshape(1, 3 * H),
      gru['Wh'], gru['bh'].reshape(1, 3 * H),
      gate['W'], gate['b'].reshape(1, H),
      dec[0]['W'], dec[0]['b'].reshape(1, H),
      dec[1]['W'], dec[1]['b'].reshape(1, 1))


def _make_sc_gather2(nchunks, chunk, wslots, epad):
    per_tile = epad // NW
    ngroups = nchunks // wslots
    assert ngroups * wslots == nchunks
    mesh = plsc.VectorSubcoreMesh(core_axis_name="c", subcore_axis_name="s",
                                  num_cores=NC, num_subcores=NS)
    out_type = [jax.ShapeDtypeStruct((epad, H), jnp.float32),
                jax.ShapeDtypeStruct((epad, H), jnp.float32)]
    scratch = [
        pltpu.VMEM((nchunks, chunk), jnp.int32),
        pltpu.VMEM((nchunks, chunk), jnp.int32),
        pltpu.VMEM((wslots * chunk, H), jnp.float32),
        pltpu.VMEM((wslots * chunk, H), jnp.float32),
    ]
    scratch += [pltpu.SemaphoreType.DMA] * (wslots + 1)

    def body(s_tab, d_tab, src_hbm, dst_hbm, outs_hbm, outd_hbm, *scr):
        srcv, dstv, sbuf, dbuf = scr[:4]
        sems_in = scr[4:4 + wslots]
        sem_out = scr[4 + wslots]
        c = lax.axis_index("c")
        s = lax.axis_index("s")
        wid = c * NS + s
        row0 = wid * nchunks
        base = wid * per_tile
        pltpu.sync_copy(src_hbm.at[pl.ds(row0, nchunks)], srcv)
        pltpu.sync_copy(dst_hbm.at[pl.ds(row0, nchunks)], dstv)

        def in_args(j, slot):
            sl = pl.ds(slot * chunk, chunk)
            return ((s_tab.at[srcv.at[j]], sbuf.at[sl], sems_in[slot]),
                    (d_tab.at[dstv.at[j]], dbuf.at[sl], sems_in[slot]))

        def out_args(j, slot):
            sl = pl.ds(slot * chunk, chunk)
            e = pl.ds(base + j * chunk, chunk)
            return ((sbuf.at[sl], outs_hbm.at[e], sem_out),
                    (dbuf.at[sl], outd_hbm.at[e], sem_out))

        for g in range(wslots):
            for a in in_args(g, g):
                pltpu.async_copy(*a)

        def group(grp, _):
            j0 = grp * wslots
            for g in range(wslots):
                j = j0 + g
                for a in in_args(j, g):
                    pltpu.make_async_copy(*a).wait()
                for a in out_args(j, g):
                    pltpu.async_copy(*a)
            for g in range(wslots):
                j = j0 + g
                for a in out_args(j, g):
                    pltpu.make_async_copy(*a).wait()

                @pl.when(grp + 1 < ngroups)
                def _(j=j, g=g):
                    for a in in_args(j + wslots, g):
                        pltpu.async_copy(*a)
            return 0
        lax.fori_loop(0, ngroups, group, 0)

    return pl.kernel(body, out_type=out_type, mesh=mesh,
                     scratch_types=scratch,
                     compiler_params=pltpu.CompilerParams(
                         use_tc_tiling_on_sc=False))


def _make_sc_scatter(nchunks, chunk, wslots, epad):
    per_tile = epad // NW
    ngroups = nchunks // wslots
    assert ngroups * wslots == nchunks
    mesh = plsc.VectorSubcoreMesh(core_axis_name="c", subcore_axis_name="s",
                                  num_cores=NC, num_subcores=NS)
    out_type = jax.ShapeDtypeStruct((NC, NPAD, H), jnp.float32)
    scratch = [
        pltpu.VMEM((nchunks, chunk), jnp.int32),
        pltpu.VMEM((wslots * chunk, H), jnp.float32),
        pltpu.VMEM((chunk, H), jnp.float32),
    ]
    scratch += [pltpu.SemaphoreType.DMA] * (wslots + 1)
    scratch.append(pltpu.VMEM_SHARED((NPAD, H), jnp.float32))

    def body(m_hbm, dst_hbm, out_hbm, *scr):
        dstv, mbuf, zbuf = scr[:3]
        sems_in = scr[3:3 + wslots]
        sem_out = scr[3 + wslots]
        acc = scr[4 + wslots]
        c = lax.axis_index("c")
        s = lax.axis_index("s")
        wid = c * NS + s
        row0 = wid * nchunks
        base = wid * per_tile
        pltpu.sync_copy(dst_hbm.at[pl.ds(row0, nchunks)], dstv)

        def zero_row(r, _):
            for g in range(H // LANES):
                zbuf[r, pl.ds(g * LANES, LANES)] = jnp.zeros(
                    (LANES,), jnp.float32)
            return 0
        lax.fori_loop(0, chunk, zero_row, 0)
        off = s * ROWS_PER_SUB
        full, rem = divmod(ROWS_PER_SUB, chunk)
        for t in range(full):
            pltpu.sync_copy(zbuf, acc.at[pl.ds(off + t * chunk, chunk)])
        if rem:
            pltpu.sync_copy(zbuf.at[pl.ds(0, rem)],
                            acc.at[pl.ds(off + full * chunk, rem)])
        plsc.subcore_barrier()

        def in_args(j, slot):
            sl = pl.ds(slot * chunk, chunk)
            return (m_hbm.at[pl.ds(base + j * chunk, chunk)], mbuf.at[sl],
                    sems_in[slot])

        def out_args(j, slot):
            sl = pl.ds(slot * chunk, chunk)
            return (mbuf.at[sl], acc.at[dstv.at[j]], sem_out)

        for g in range(wslots):
            pltpu.async_copy(*in_args(g, g))

        def group(grp, _):
            j0 = grp * wslots
            for g in range(wslots):
                j = j0 + g
                pltpu.make_async_copy(*in_args(j, g)).wait()
                pltpu.async_copy(*out_args(j, g), add=True)
            for g in range(wslots):
                j = j0 + g
                pltpu.make_async_copy(*out_args(j, g)).wait()

                @pl.when(grp + 1 < ngroups)
                def _(j=j, g=g):
                    pltpu.async_copy(*in_args(j + wslots, g))
            return 0
        lax.fori_loop(0, ngroups, group, 0)

        plsc.subcore_barrier()
        sl = pl.ds(s * ROWS_PER_SUB, ROWS_PER_SUB)
        pltpu.sync_copy(acc.at[sl], out_hbm.at[c].at[sl])

    return pl.kernel(body, out_type=out_type, mesh=mesh,
                     scratch_types=scratch,
                     compiler_params=pltpu.CompilerParams(
                         use_tc_tiling_on_sc=False))


@functools.cache
def _sc_kernels():
    return {
        'gather_big': _make_sc_gather2(40, 128, 4, EPAD),
        'gather_small': _make_sc_gather2(5, 64, 5, ECPAD),
        'scatter_big': _make_sc_scatter(40, 128, 4, EPAD),
        'scatter_small': _make_sc_scatter(5, 64, 5, ECPAD),
    }


def _pad_rows(x, rows):
    return jnp.pad(x, ((0, rows - x.shape[0]), (0, 0)))


def _pad_idx(ei, epad, chunk):
    src = jnp.pad(ei[0], (0, epad - ei.shape[1]))
    dst = jnp.pad(ei[1], (0, epad - ei.shape[1]),
                  constant_values=DUMMY_DST)
    return src.reshape(-1, chunk), dst.reshape(-1, chunk)


def kernel(params, x_1d, x_2d, edge_index_pipe, edge_attr_pipe,
           edge_index_surface, edge_attr_surface, edge_index_c12,
           edge_attr_c12, edge_index_c21, edge_attr_c21, h_1d, h_2d):
    sck = _sc_kernels()

    src_p, dst_p = _pad_idx(edge_index_pipe, EPAD, 128)
    src_s, dst_s = _pad_idx(edge_index_surface, EPAD, 128)
    src_12, dst_12 = _pad_idx(edge_index_c12, ECPAD, 64)
    src_21, dst_21 = _pad_idx(edge_index_c21, ECPAD, 64)

    x1 = _tc_encode(_pad_rows(x_1d, NPAD), params['enc_1d'], tm=128)
    x2 = _tc_encode(_pad_rows(x_2d, NPAD), params['enc_2d'], tm=128)
    ep = _tc_encode(_pad_rows(edge_attr_pipe, EPAD), params['enc_pipe'])
    es = _tc_encode(_pad_rows(edge_attr_surface, EPAD), params['enc_surf'])
    e12 = _tc_encode(_pad_rows(edge_attr_c12, ECPAD), params['enc_c12'],
                     tm=128)
    e21 = _tc_encode(_pad_rows(edge_attr_c21, ECPAD), params['enc_c21'],
                     tm=128)

    for lp in params['layers']:
        xs_p, xd_p = sck['gather_big'](x1, x1, src_p, dst_p)
        xs_s, xd_s = sck['gather_big'](x2, x2, src_s, dst_s)
        xs_12, xd_12 = sck['gather_small'](x1, x2, src_12, dst_12)
        xs_21, xd_21 = sck['gather_small'](x2, x1, src_21, dst_21)

        m_p = _tc_edge_mlp(xs_p, ep, xd_p, lp['conv_pipe'], False)
        m_s = _tc_edge_mlp(xs_s, es, xd_s, lp['conv_surf'], False)
        m_12 = _tc_edge_mlp(xs_12, e12, xd_12, lp['conv_c12'], False,
                            tm=128)
        m_21 = _tc_edge_mlp(xs_21, e21, xd_21, lp['conv_c21'], False,
                            tm=128)

        part_p = sck['scatter_big'](m_p, dst_p)
        part_s = sck['scatter_big'](m_s, dst_s)
        part_12 = sck['scatter_small'](m_12, dst_12)
        part_21 = sck['scatter_small'](m_21, dst_21)

        x1 = _tc_tail(x1, part_p, part_21, lp['ln_1d'])
        x2 = _tc_tail(x2, part_s, part_12, lp['ln_2d'])

        us_p, ud_p = sck['gather_big'](x1, x1, src_p, dst_p)
        us_s, ud_s = sck['gather_big'](x2, x2, src_s, dst_s)
        ep = _tc_edge_mlp(us_p, ep, ud_p, lp['eu_pipe'], True)
        es = _tc_edge_mlp(us_s, es, ud_s, lp['eu_surf'], True)

    h1n, out1 = _tc_head(x1, _pad_rows(h_1d, NPAD), params['gru_1d'],
                         params['gate_1d'], params['dec_1d'])
    h2n, out2 = _tc_head(x2, _pad_rows(h_2d, NPAD), params['gru_2d'],
                         params['gate_2d'], params['dec_2d'])

    return (out1[:N_NODES], out2[:N_NODES],
            h1n[:N_NODES], h2n[:N_NODES])


# speedup vs baseline: 1.3598x; 1.1834x over previous
import functools

import jax
import jax.numpy as jnp
from jax import lax
from jax.experimental import pallas as pl
from jax.experimental.pallas import tpu as pltpu
from jax.experimental.pallas import tpu_sc as plsc

H = 64
N_NODES = 10000
NPAD = 10112
EPAD = 163840
ECPAD = 10240
NC, NS, LANES = 2, 16, 16
NW = NC * NS
DUMMY_DST = N_NODES
ROWS_PER_SUB = NPAD // NS


def _enc_kern(x_ref, w1_ref, b1_ref, w2_ref, b2_ref, o_ref):
    h = jnp.dot(x_ref[...], w1_ref[...],
                preferred_element_type=jnp.float32) + b1_ref[...]
    h = jax.nn.gelu(h)
    o_ref[...] = jnp.dot(h, w2_ref[...],
                         preferred_element_type=jnp.float32) + b2_ref[...]


def _tc_encode(x, mlp, tm=512):
    m, k = x.shape
    h1 = mlp[0]['W'].shape[1]
    h2 = mlp[1]['W'].shape[1]
    tm = min(tm, m)
    return pl.pallas_call(
        _enc_kern, grid=(m // tm,),
        in_specs=[pl.BlockSpec((tm, k), lambda i: (i, 0)),
                  pl.BlockSpec((k, h1), lambda i: (0, 0)),
                  pl.BlockSpec((1, h1), lambda i: (0, 0)),
                  pl.BlockSpec((h1, h2), lambda i: (0, 0)),
                  pl.BlockSpec((1, h2), lambda i: (0, 0))],
        out_specs=pl.BlockSpec((tm, h2), lambda i: (i, 0)),
        out_shape=jax.ShapeDtypeStruct((m, h2), jnp.float32),
    )(x, mlp[0]['W'], mlp[0]['b'].reshape(1, h1),
      mlp[1]['W'], mlp[1]['b'].reshape(1, h2))


def _edge_mlp_kern(residual, xs_ref, ee_ref, xd_ref, w1_ref, b1_ref,
                   w2_ref, b2_ref, o_ref):
    cat = jnp.concatenate([xs_ref[...], ee_ref[...], xd_ref[...]], axis=-1)
    h = jnp.dot(cat, w1_ref[...],
                preferred_element_type=jnp.float32) + b1_ref[...]
    h = jax.nn.gelu(h)
    m = jnp.dot(h, w2_ref[...],
                preferred_element_type=jnp.float32) + b2_ref[...]
    if residual:
        m = ee_ref[...] + m
    o_ref[...] = m


def _tc_edge_mlp(xs, ee, xd, mlp, residual, tm=1024):
    m = xs.shape[0]
    return pl.pallas_call(
        functools.partial(_edge_mlp_kern, residual), grid=(m // tm,),
        in_specs=[pl.BlockSpec((tm, H), lambda i: (i, 0)),
                  pl.BlockSpec((tm, H), lambda i: (i, 0)),
                  pl.BlockSpec((tm, H), lambda i: (i, 0)),
                  pl.BlockSpec((3 * H, H), lambda i: (0, 0)),
                  pl.BlockSpec((1, H), lambda i: (0, 0)),
                  pl.BlockSpec((H, H), lambda i: (0, 0)),
                  pl.BlockSpec((1, H), lambda i: (0, 0))],
        out_specs=pl.BlockSpec((tm, H), lambda i: (i, 0)),
        out_shape=jax.ShapeDtypeStruct((m, H), jnp.float32),
    )(xs, ee, xd, mlp[0]['W'], mlp[0]['b'].reshape(1, H),
      mlp[1]['W'], mlp[1]['b'].reshape(1, H))


def _tail_kern(x_ref, pa_ref, pb_ref, g_ref, b_ref, o_ref):
    a = (pa_ref[0] + pa_ref[1]) + (pb_ref[0] + pb_ref[1])
    t = x_ref[...] + a
    mu = jnp.mean(t, axis=-1, keepdims=True)
    v = jnp.mean((t - mu) ** 2, axis=-1, keepdims=True)
    o_ref[...] = (t - mu) / jnp.sqrt(v + 1e-5) * g_ref[...] + b_ref[...]


def _tc_tail(x, part_a, part_b, ln, tm=128):
    m = x.shape[0]
    return pl.pallas_call(
        _tail_kern, grid=(m // tm,),
        in_specs=[pl.BlockSpec((tm, H), lambda i: (i, 0)),
                  pl.BlockSpec((2, tm, H), lambda i: (0, i, 0)),
                  pl.BlockSpec((2, tm, H), lambda i: (0, i, 0)),
                  pl.BlockSpec((1, H), lambda i: (0, 0)),
                  pl.BlockSpec((1, H), lambda i: (0, 0))],
        out_specs=pl.BlockSpec((tm, H), lambda i: (i, 0)),
        out_shape=jax.ShapeDtypeStruct((m, H), jnp.float32),
    )(x, part_a, part_b, ln['g'].reshape(1, H), ln['b'].reshape(1, H))


def _head_kern(x_ref, h_ref, wi_ref, bi_ref, wh_ref, bh_ref,
               wg_ref, bg_ref, v1_ref, c1_ref, v2_ref, c2_ref,
               hn_ref, o_ref):
    x = x_ref[...]
    h = h_ref[...]
    gx = jnp.dot(x, wi_ref[...], preferred_element_type=jnp.float32) \
        + bi_ref[...]
    gh = jnp.dot(h, wh_ref[...], preferred_element_type=jnp.float32) \
        + bh_ref[...]
    r = jax.nn.sigmoid(gx[:, 0:H] + gh[:, 0:H])
    z = jax.nn.sigmoid(gx[:, H:2 * H] + gh[:, H:2 * H])
    n = jnp.tanh(gx[:, 2 * H:3 * H] + r * gh[:, 2 * H:3 * H])
    hn = (1.0 - z) * n + z * h
    g = jax.nn.sigmoid(
        jnp.dot(jnp.concatenate([x, hn], axis=-1), wg_ref[...],
                preferred_element_type=jnp.float32) + bg_ref[...])
    xo = (1.0 - g) * x + g * hn
    d = jax.nn.gelu(
        jnp.dot(xo, v1_ref[...], preferred_element_type=jnp.float32)
        + c1_ref[...])
    o_ref[...] = jnp.dot(d, v2_ref[...],
                         preferred_element_type=jnp.float32) + c2_ref[...]
    hn_ref[...] = hn


def _tc_head(x, h, gru, gate, dec, tm=128):
    m = x.shape[0]
    full = lambda shape: pl.BlockSpec(shape, lambda i: tuple(0 for _ in shape))
    return pl.pallas_call(
        _head_kern, grid=(m // tm,),
        in_specs=[pl.BlockSpec((tm, H), lambda i: (i, 0)),
                  pl.BlockSpec((tm, H), lambda i: (i, 0)),
                  full((H, 3 * H)), full((1, 3 * H)),
                  full((H, 3 * H)), full((1, 3 * H)),
                  full((2 * H, H)), full((1, H)),
                  full((H, H)), full((1, H)), full((H, 1)), full((1, 1))],
        out_specs=[pl.BlockSpec((tm, H), lambda i: (i, 0)),
                   pl.BlockSpec((tm, 1), lambda i: (i, 0))],
        out_shape=[jax.ShapeDtypeStruct((m, H), jnp.float32),
                   jax.ShapeDtypeStruct((m, 1), jnp.float32)],
    )(x, h, gru['Wi'], gru['bi'].reshape(1, 3 * H),
      gru['Wh'], gru['bh'].reshape(1, 3 * H),
      gate['W'], gate['b'].reshape(1, H),
      dec[0]['W'], dec[0]['b'].reshape(1, H),
      dec[1]['W'], dec[1]['b'].reshape(1, 1))


def _make_sc_gather2(nchunks, chunk, wslots, epad):
    per_tile = epad // NW
    ngroups = nchunks // wslots
    assert ngroups * wslots == nchunks
    mesh = plsc.VectorSubcoreMesh(core_axis_name="c", subcore_axis_name="s",
                                  num_cores=NC, num_subcores=NS)
    out_type = [jax.ShapeDtypeStruct((epad, H), jnp.float32),
                jax.ShapeDtypeStruct((epad, H), jnp.float32)]
    scratch = [
        pltpu.VMEM((nchunks, chunk), jnp.int32),
        pltpu.VMEM((nchunks, chunk), jnp.int32),
        pltpu.VMEM((wslots * chunk, H), jnp.float32),
        pltpu.VMEM((wslots * chunk, H), jnp.float32),
    ]
    scratch += [pltpu.SemaphoreType.DMA] * (wslots + 1)

    def body(s_tab, d_tab, src_hbm, dst_hbm, outs_hbm, outd_hbm, *scr):
        srcv, dstv, sbuf, dbuf = scr[:4]
        sems_in = scr[4:4 + wslots]
        sem_out = scr[4 + wslots]
        c = lax.axis_index("c")
        s = lax.axis_index("s")
        wid = c * NS + s
        row0 = wid * nchunks
        base = wid * per_tile
        pltpu.sync_copy(src_hbm.at[pl.ds(row0, nchunks)], srcv)
        pltpu.sync_copy(dst_hbm.at[pl.ds(row0, nchunks)], dstv)

        def in_args(j, slot):
            sl = pl.ds(slot * chunk, chunk)
            return ((s_tab.at[srcv.at[j]], sbuf.at[sl], sems_in[slot]),
                    (d_tab.at[dstv.at[j]], dbuf.at[sl], sems_in[slot]))

        def out_args(j, slot):
            sl = pl.ds(slot * chunk, chunk)
            e = pl.ds(base + j * chunk, chunk)
            return ((sbuf.at[sl], outs_hbm.at[e], sem_out),
                    (dbuf.at[sl], outd_hbm.at[e], sem_out))

        for g in range(wslots):
            for a in in_args(g, g):
                pltpu.async_copy(*a)

        def group(grp, _):
            j0 = grp * wslots
            for g in range(wslots):
                j = j0 + g
                for a in in_args(j, g):
                    pltpu.make_async_copy(*a).wait()
                for a in out_args(j, g):
                    pltpu.async_copy(*a)
            for g in range(wslots):
                j = j0 + g
                for a in out_args(j, g):
                    pltpu.make_async_copy(*a).wait()

                @pl.when(grp + 1 < ngroups)
                def _(j=j, g=g):
                    for a in in_args(j + wslots, g):
                        pltpu.async_copy(*a)
            return 0
        lax.fori_loop(0, ngroups, group, 0)

    return pl.kernel(body, out_type=out_type, mesh=mesh,
                     scratch_types=scratch,
                     compiler_params=pltpu.CompilerParams(
                         use_tc_tiling_on_sc=False))


def _make_sc_scatter(nchunks, chunk, wslots, epad):
    per_tile = epad // NW
    ngroups = nchunks // wslots
    assert ngroups * wslots == nchunks
    mesh = plsc.VectorSubcoreMesh(core_axis_name="c", subcore_axis_name="s",
                                  num_cores=NC, num_subcores=NS)
    out_type = jax.ShapeDtypeStruct((NC, NPAD, H), jnp.float32)
    scratch = [
        pltpu.VMEM((nchunks, chunk), jnp.int32),
        pltpu.VMEM((wslots * chunk, H), jnp.float32),
        pltpu.VMEM((chunk, H), jnp.float32),
    ]
    scratch += [pltpu.SemaphoreType.DMA] * (wslots + 1)
    scratch.append(pltpu.VMEM_SHARED((NPAD, H), jnp.float32))

    def body(m_hbm, dst_hbm, out_hbm, *scr):
        dstv, mbuf, zbuf = scr[:3]
        sems_in = scr[3:3 + wslots]
        sem_out = scr[3 + wslots]
        acc = scr[4 + wslots]
        c = lax.axis_index("c")
        s = lax.axis_index("s")
        wid = c * NS + s
        row0 = wid * nchunks
        base = wid * per_tile
        pltpu.sync_copy(dst_hbm.at[pl.ds(row0, nchunks)], dstv)

        def zero_row(r, _):
            for g in range(H // LANES):
                zbuf[r, pl.ds(g * LANES, LANES)] = jnp.zeros(
                    (LANES,), jnp.float32)
            return 0
        lax.fori_loop(0, chunk, zero_row, 0)
        off = s * ROWS_PER_SUB
        full, rem = divmod(ROWS_PER_SUB, chunk)
        for t in range(full):
            pltpu.sync_copy(zbuf, acc.at[pl.ds(off + t * chunk, chunk)])
        if rem:
            pltpu.sync_copy(zbuf.at[pl.ds(0, rem)],
                            acc.at[pl.ds(off + full * chunk, rem)])
        plsc.subcore_barrier()

        def in_args(j, slot):
            sl = pl.ds(slot * chunk, chunk)
            return (m_hbm.at[pl.ds(base + j * chunk, chunk)], mbuf.at[sl],
                    sems_in[slot])

        def out_args(j, slot):
            sl = pl.ds(slot * chunk, chunk)
            return (mbuf.at[sl], acc.at[dstv.at[j]], sem_out)

        for g in range(wslots):
            pltpu.async_copy(*in_args(g, g))

        def group(grp, _):
            j0 = grp * wslots
            for g in range(wslots):
                j = j0 + g
                pltpu.make_async_copy(*in_args(j, g)).wait()
                pltpu.async_copy(*out_args(j, g), add=True)
            for g in range(wslots):
                j = j0 + g
                pltpu.make_async_copy(*out_args(j, g)).wait()

                @pl.when(grp + 1 < ngroups)
                def _(j=j, g=g):
                    pltpu.async_copy(*in_args(j + wslots, g))
            return 0
        lax.fori_loop(0, ngroups, group, 0)

        plsc.subcore_barrier()
        sl = pl.ds(s * ROWS_PER_SUB, ROWS_PER_SUB)
        pltpu.sync_copy(acc.at[sl], out_hbm.at[c].at[sl])

    return pl.kernel(body, out_type=out_type, mesh=mesh,
                     scratch_types=scratch,
                     compiler_params=pltpu.CompilerParams(
                         use_tc_tiling_on_sc=False))


@functools.cache
def _sc_kernels():
    return {
        'gather_big': _make_sc_gather2(40, 128, 4, EPAD),
        'gather_small': _make_sc_gather2(5, 64, 5, ECPAD),
        'scatter_big': _make_sc_scatter(40, 128, 4, EPAD),
        'scatter_small': _make_sc_scatter(5, 64, 5, ECPAD),
    }


def _pad_rows(x, rows):
    return jnp.pad(x, ((0, rows - x.shape[0]), (0, 0)))


def _pad_idx(ei, epad, chunk):
    src = jnp.pad(ei[0], (0, epad - ei.shape[1]))
    dst = jnp.pad(ei[1], (0, epad - ei.shape[1]),
                  constant_values=DUMMY_DST)
    return src.reshape(-1, chunk), dst.reshape(-1, chunk)


def kernel(params, x_1d, x_2d, edge_index_pipe, edge_attr_pipe,
           edge_index_surface, edge_attr_surface, edge_index_c12,
           edge_attr_c12, edge_index_c21, edge_attr_c21, h_1d, h_2d):
    sck = _sc_kernels()

    src_p, dst_p = _pad_idx(edge_index_pipe, EPAD, 128)
    src_s, dst_s = _pad_idx(edge_index_surface, EPAD, 128)
    src_12, dst_12 = _pad_idx(edge_index_c12, ECPAD, 64)
    src_21, dst_21 = _pad_idx(edge_index_c21, ECPAD, 64)

    x1 = _tc_encode(_pad_rows(x_1d, NPAD), params['enc_1d'], tm=128)
    x2 = _tc_encode(_pad_rows(x_2d, NPAD), params['enc_2d'], tm=128)
    ep = _tc_encode(_pad_rows(edge_attr_pipe, EPAD), params['enc_pipe'])
    es = _tc_encode(_pad_rows(edge_attr_surface, EPAD), params['enc_surf'])
    e12 = _tc_encode(_pad_rows(edge_attr_c12, ECPAD), params['enc_c12'],
                     tm=128)
    e21 = _tc_encode(_pad_rows(edge_attr_c21, ECPAD), params['enc_c21'],
                     tm=128)

    for lp in params['layers']:
        xs_p, xd_p = sck['gather_big'](x1, x1, src_p, dst_p)
        xs_s, xd_s = sck['gather_big'](x2, x2, src_s, dst_s)
        xs_12, xd_12 = sck['gather_small'](x1, x2, src_12, dst_12)
        xs_21, xd_21 = sck['gather_small'](x2, x1, src_21, dst_21)

        m_p = _tc_edge_mlp(xs_p, ep, xd_p, lp['conv_pipe'], False)
        m_s = _tc_edge_mlp(xs_s, es, xd_s, lp['conv_surf'], False)
        m_12 = _tc_edge_mlp(xs_12, e12, xd_12, lp['conv_c12'], False,
                            tm=128)
        m_21 = _tc_edge_mlp(xs_21, e21, xd_21, lp['conv_c21'], False,
                            tm=128)

        part_p = sck['scatter_big'](m_p, dst_p)
        part_s = sck['scatter_big'](m_s, dst_s)
        part_12 = sck['scatter_small'](m_12, dst_12)
        part_21 = sck['scatter_small'](m_21, dst_21)

        x1 = _tc_tail(x1, part_p, part_21, lp['ln_1d'])
        x2 = _tc_tail(x2, part_s, part_12, lp['ln_2d'])

        us_p, ud_p = sck['gather_big'](x1, x1, src_p, dst_p)
        us_s, ud_s = sck['gather_big'](x2, x2, src_s, dst_s)
        ep = _tc_edge_mlp(us_p, ep, ud_p, lp['eu_pipe'], True)
        es = _tc_edge_mlp(us_s, es, ud_s, lp['eu_surf'], True)

    h1n, out1 = _tc_head(x1, _pad_rows(h_1d, NPAD), params['gru_1d'],
                         params['gate_1d'], params['dec_1d'])
    h2n, out2 = _tc_head(x2, _pad_rows(h_2d, NPAD), params['gru_2d'],
                         params['gate_2d'], params['dec_2d'])

    return (out1[:N_NODES], out2[:N_NODES],
            h1n[:N_NODES], h2n[:N_NODES])
